# Initial kernel scaffold; baseline (speedup 1.0000x reference)
import jax
import jax.numpy as jnp
from jax.experimental import pallas as pl


def kernel(x, adj_t, W1, b1, W2, b2):
    raise NotImplementedError("write your pallas kernel here")


# speedup vs baseline: 6.6854x; 6.6854x over previous
import functools

import jax
import jax.numpy as jnp
from jax import lax
from jax.experimental import pallas as pl
from jax.experimental.pallas import tpu as pltpu
from jax.experimental.pallas import tpu_sc as plsc

N = 10000
D = 256
DH = 128
NROW = 10240
E = 160000
CHUNK = 128
NCHUNK = 80
EPAD = 16 * NCHUNK * CHUNK
TRASH = 10200
BN = 1024
NSUB = 16
RPS = NROW // NSUB

_MESH = dict(core_axis_name="c", subcore_axis_name="s")


def _sc_degree(dst3, ones_chunk, zeros16):
    mesh = plsc.VectorSubcoreMesh(**_MESH)

    @functools.partial(
        pl.kernel,
        mesh=mesh,
        out_type=jax.ShapeDtypeStruct((2, NROW, 16), jnp.float32),
        scratch_types=[
            pltpu.VMEM((NCHUNK, CHUNK), jnp.int32),
            pltpu.VMEM((CHUNK, 16), jnp.float32),
            pltpu.VMEM_SHARED((NROW, 16), jnp.float32),
        ],
    )
    def k(dst_hbm, ones_hbm, z_hbm, out_hbm, dst_v, ones_v, acc):
        cid = lax.axis_index("c")
        sid = lax.axis_index("s")
        pltpu.sync_copy(z_hbm.at[pl.ds(sid * RPS, RPS)],
                        acc.at[pl.ds(sid * RPS, RPS)])
        pltpu.sync_copy(dst_hbm.at[sid], dst_v)
        pltpu.sync_copy(ones_hbm, ones_v)
        plsc.subcore_barrier()

        half = NCHUNK // 2

        @pl.loop(cid * half, (cid + 1) * half)
        def _(j):
            pltpu.sync_copy(ones_v, acc.at[dst_v.at[j]], add=True)

        plsc.subcore_barrier()
        pltpu.sync_copy(acc.at[pl.ds(sid * RPS, RPS)],
                        out_hbm.at[cid].at[pl.ds(sid * RPS, RPS)])

    return k(dst3, ones_chunk, zeros16)


def _sc_scatter(g, src3, dst3, zeros):
    mesh = plsc.VectorSubcoreMesh(**_MESH)

    @functools.partial(
        pl.kernel,
        mesh=mesh,
        out_type=jax.ShapeDtypeStruct((2, NROW, DH), jnp.float32),
        scratch_types=[
            pltpu.VMEM((NCHUNK, CHUNK), jnp.int32),
            pltpu.VMEM((NCHUNK, CHUNK), jnp.int32),
            pltpu.VMEM((CHUNK, DH), jnp.float32),
            pltpu.VMEM_SHARED((NROW, DH), jnp.float32),
            pltpu.SemaphoreType.DMA,
        ],
    )
    def k(g_hbm, src_hbm, dst_hbm, z_hbm, out_hbm,
          src_v, dst_v, rows, acc, sem):
        cid = lax.axis_index("c")
        sid = lax.axis_index("s")
        pltpu.sync_copy(z_hbm.at[pl.ds(sid * RPS, RPS)],
                        acc.at[pl.ds(sid * RPS, RPS)])
        pltpu.sync_copy(src_hbm.at[sid], src_v)
        pltpu.sync_copy(dst_hbm.at[sid], dst_v)
        plsc.subcore_barrier()

        gtab = g_hbm.at[cid]

        @pl.loop(0, NCHUNK)
        def _(j):
            pltpu.async_copy(gtab.at[src_v.at[j]], rows, sem).wait()
            pltpu.sync_copy(rows, acc.at[dst_v.at[j]], add=True)

        plsc.subcore_barrier()
        pltpu.sync_copy(acc.at[pl.ds(sid * RPS, RPS)],
                        out_hbm.at[cid].at[pl.ds(sid * RPS, RPS)])

    return k(g, src3, dst3, zeros)


def _tc_matmul_split(x, w_split):
    def body(x_ref, w_ref, o_ref):
        o_ref[0] = jnp.dot(x_ref[...], w_ref[0],
                           preferred_element_type=jnp.float32,
                           precision=lax.Precision.HIGHEST)

    return pl.pallas_call(
        body,
        grid=(NROW // BN, 2),
        in_specs=[pl.BlockSpec((BN, D), lambda i, c: (i, 0)),
                  pl.BlockSpec((1, D, DH), lambda i, c: (c, 0, 0))],
        out_specs=pl.BlockSpec((1, BN, DH), lambda i, c: (c, i, 0)),
        out_shape=jax.ShapeDtypeStruct((2, NROW, DH), jnp.float32),
    )(x, w_split)


def _tc_scale(deg_parts, h):
    def body(dg_ref, h_ref, di_ref, g_ref):
        d = dg_ref[0, :, 0:1] + dg_ref[1, :, 0:1] + 1.0
        di = lax.rsqrt(d)
        di_ref[...] = di
        g_ref[0] = h_ref[0] * di
        g_ref[1] = h_ref[1] * di

    return pl.pallas_call(
        body,
        grid=(NROW // BN,),
        in_specs=[pl.BlockSpec((2, BN, 16), lambda i: (0, i, 0)),
                  pl.BlockSpec((2, BN, DH), lambda i: (0, i, 0))],
        out_specs=[pl.BlockSpec((BN, 1), lambda i: (i, 0)),
                   pl.BlockSpec((2, BN, DH), lambda i: (0, i, 0))],
        out_shape=[jax.ShapeDtypeStruct((NROW, 1), jnp.float32),
                   jax.ShapeDtypeStruct((2, NROW, DH), jnp.float32)],
    )(deg_parts, h)


def _tc_layer2(s1, g1, di, b1_split, w2_split):
    def body(s_ref, g_ref, di_ref, b_ref, w_ref, o_ref):
        d = di_ref[...]
        t0 = jax.nn.relu((s_ref[0] + g_ref[0]) * d + b_ref[0])
        t1 = jax.nn.relu((s_ref[1] + g_ref[1]) * d + b_ref[1])
        t = jnp.concatenate([t0, t1], axis=1)
        o_ref[0] = jnp.dot(t, w_ref[0],
                           preferred_element_type=jnp.float32,
                           precision=lax.Precision.HIGHEST) * d

    return pl.pallas_call(
        body,
        grid=(NROW // BN, 2),
        in_specs=[pl.BlockSpec((2, BN, DH), lambda i, c: (0, i, 0)),
                  pl.BlockSpec((2, BN, DH), lambda i, c: (0, i, 0)),
                  pl.BlockSpec((BN, 1), lambda i, c: (i, 0)),
                  pl.BlockSpec((2, 1, DH), lambda i, c: (0, 0, 0)),
                  pl.BlockSpec((1, D, DH), lambda i, c: (c, 0, 0))],
        out_specs=pl.BlockSpec((1, BN, DH), lambda i, c: (c, i, 0)),
        out_shape=jax.ShapeDtypeStruct((2, NROW, DH), jnp.float32),
    )(s1, g1, di, b1_split, w2_split)


def _tc_final(s2, g2, di, b2_split):
    def body(s_ref, g_ref, di_ref, b_ref, o_ref):
        d = di_ref[...]
        o0 = (s_ref[0] + g_ref[0]) * d + b_ref[0]
        o1 = (s_ref[1] + g_ref[1]) * d + b_ref[1]
        o_ref[...] = jnp.concatenate([o0, o1], axis=1)

    return pl.pallas_call(
        body,
        grid=(NROW // BN,),
        in_specs=[pl.BlockSpec((2, BN, DH), lambda i: (0, i, 0)),
                  pl.BlockSpec((2, BN, DH), lambda i: (0, i, 0)),
                  pl.BlockSpec((BN, 1), lambda i: (i, 0)),
                  pl.BlockSpec((2, 1, DH), lambda i: (0, 0, 0))],
        out_specs=pl.BlockSpec((BN, D), lambda i: (i, 0)),
        out_shape=jax.ShapeDtypeStruct((NROW, D), jnp.float32),
    )(s2, g2, di, b2_split)


def kernel(x, adj_t, W1, b1, W2, b2):
    src = adj_t[0].astype(jnp.int32)
    dst = adj_t[1].astype(jnp.int32)
    pad = EPAD - E
    src_p = jnp.concatenate([src, jnp.zeros((pad,), jnp.int32)])
    dst_p = jnp.concatenate([dst, jnp.full((pad,), TRASH, jnp.int32)])
    src3 = src_p.reshape(NSUB, NCHUNK, CHUNK)
    dst3 = dst_p.reshape(NSUB, NCHUNK, CHUNK)

    x_p = jnp.pad(x, ((0, NROW - N), (0, 0)))
    w1s = W1.reshape(D, 2, DH).transpose(1, 0, 2)
    w2s = W2.reshape(D, 2, DH).transpose(1, 0, 2)
    b1s = b1.reshape(2, 1, DH)
    b2s = b2.reshape(2, 1, DH)
    zeros_big = jnp.zeros((NROW, DH), jnp.float32)
    zeros16 = jnp.zeros((NROW, 16), jnp.float32)
    ones_chunk = jnp.ones((CHUNK, 16), jnp.float32)

    deg_parts = _sc_degree(dst3, ones_chunk, zeros16)
    h1 = _tc_matmul_split(x_p, w1s)
    di, g1 = _tc_scale(deg_parts, h1)
    s1 = _sc_scatter(g1, src3, dst3, zeros_big)
    g2 = _tc_layer2(s1, g1, di, b1s, w2s)
    s2 = _sc_scatter(g2, src3, dst3, zeros_big)
    out = _tc_final(s2, g2, di, b2s)
    return out[:N]
